# speedup vs baseline: 5.5786x; 5.5786x over previous
import functools

import jax
import jax.numpy as jnp
from jax.experimental import pallas as pl
from jax.experimental.pallas import tpu as pltpu

B = 1024
T = 50
H = 128
STAT = 8
MULTI = 8
TGT = 4
NCAT = 3
VOCAB = 100
VPAD = 104

BB = 32
RB = BB * T


def _tft_body(me_ref, tgt_ref, stat_ref, sv_ref, sb_ref, mv_ref, mb_ref,
              tv_ref, tb_ref, e0_ref, e1_ref, e2_ref,
              s_out, k_out, t_out,
              ghi0, glo0, ghi1, glo1, ghi2, glo2):
    i = pl.program_id(0)

    @pl.when(i == 0)
    def _():
        for e_ref, ghi, glo in ((e0_ref, ghi0, glo0),
                                (e1_ref, ghi1, glo1),
                                (e2_ref, ghi2, glo2)):
            e = e_ref[...]
            g = 0.5 * e * (1.0 + jax.lax.erf(e * 0.7071067811865476))
            hi = g.astype(jnp.bfloat16)
            ghi[...] = hi
            glo[...] = (g - hi.astype(jnp.float32)).astype(jnp.bfloat16)

    stat = stat_ref[...]
    s_out[...] = stat[:, :, None] * sv_ref[...][None] + sb_ref[...][None]

    tgt = tgt_ref[...]
    t_out[...] = tgt[:, :, None] * tv_ref[...][None] + tb_ref[...][None]

    me = me_ref[...]
    mv3 = mv_ref[NCAT:NCAT + 1, :]
    k_out[:, NCAT:, :] = (me[:, NCAT:, None] * mv3[None]
                          + mb_ref[...][None, NCAT:, :])

    iota = jax.lax.broadcasted_iota(jnp.int32, (1, VPAD), 1)
    for c, (ghi, glo) in enumerate(((ghi0, glo0), (ghi1, glo1), (ghi2, glo2))):
        idx = me[:, c:c + 1].astype(jnp.int32)
        oh = (idx == iota).astype(jnp.bfloat16)
        rows = jax.lax.dot_general(
            oh, ghi[...], (((1,), (0,)), ((), ())),
            preferred_element_type=jnp.float32)
        rows = rows + jax.lax.dot_general(
            oh, glo[...], (((1,), (0,)), ((), ())),
            preferred_element_type=jnp.float32)
        k_out[:, c:c + 1, :] = rows[:, None, :]


@jax.jit
def kernel(target_inp, stat_exog, multi_exog, stat_vec, stat_bias, multi_vec,
           multi_bias, tgt_vec, tgt_bias, emb0, emb1, emb2):
    me2 = multi_exog.reshape(B * T, MULTI)
    tgt2 = target_inp.reshape(B * T, TGT)
    pad = jnp.zeros((VPAD - VOCAB, H), jnp.float32)
    e0 = jnp.concatenate([emb0, pad], axis=0)
    e1 = jnp.concatenate([emb1, pad], axis=0)
    e2 = jnp.concatenate([emb2, pad], axis=0)

    nsteps = B // BB
    full = lambda shape: pl.BlockSpec(shape, lambda i: (0,) * len(shape))

    s3, k3, t3 = pl.pallas_call(
        _tft_body,
        grid=(nsteps,),
        in_specs=[
            pl.BlockSpec((RB, MULTI), lambda i: (i, 0)),
            pl.BlockSpec((RB, TGT), lambda i: (i, 0)),
            pl.BlockSpec((BB, STAT), lambda i: (i, 0)),
            full((STAT, H)), full((STAT, H)),
            full((MULTI, H)), full((MULTI, H)),
            full((TGT, H)), full((TGT, H)),
            full((VPAD, H)), full((VPAD, H)), full((VPAD, H)),
        ],
        out_specs=[
            pl.BlockSpec((BB, STAT, H), lambda i: (i, 0, 0)),
            pl.BlockSpec((RB, MULTI, H), lambda i: (i, 0, 0)),
            pl.BlockSpec((RB, TGT, H), lambda i: (i, 0, 0)),
        ],
        out_shape=[
            jax.ShapeDtypeStruct((B, STAT, H), jnp.float32),
            jax.ShapeDtypeStruct((B * T, MULTI, H), jnp.float32),
            jax.ShapeDtypeStruct((B * T, TGT, H), jnp.float32),
        ],
        scratch_shapes=[pltpu.VMEM((VPAD, H), jnp.bfloat16)] * 6,
    )(me2, tgt2, stat_exog, stat_vec, stat_bias, multi_vec, multi_bias,
      tgt_vec, tgt_bias, e0, e1, e2)

    return (s3, k3.reshape(B, T, MULTI, H), t3.reshape(B, T, TGT, H))
